# baseline (speedup 1.0000x reference)
import functools

import jax
import jax.numpy as jnp
from jax import lax
from jax.experimental import pallas as pl
from jax.experimental.pallas import tpu as pltpu
from jax.experimental.pallas import tpu_sc as plsc

NF = 32
L = 16
NC = 2
NS = 16
NW = NC * NS
CH = 128


def _body(uc_hbm, ic_hbm, uf_hbm, if_hbm, params_hbm, out_hbm,
          uidx_v, iidx_v, urows_v, irows_v, params_v, out_v, sem,
          *, bpw):
    nchunk = bpw // CH
    wid = lax.axis_index("s") * NC + lax.axis_index("c")
    base = wid * bpw

    for j in range(nchunk):
        pltpu.sync_copy(uc_hbm.at[pl.ds(base + j * CH, CH)], uidx_v.at[j])
        pltpu.sync_copy(ic_hbm.at[pl.ds(base + j * CH, CH)], iidx_v.at[j])
    pltpu.sync_copy(params_hbm, params_v)

    copies = []
    for j in range(nchunk):
        copies.append(pltpu.async_copy(
            uf_hbm.at[uidx_v.at[j]], urows_v.at[pl.ds(j * CH, CH)], sem))
        copies.append(pltpu.async_copy(
            if_hbm.at[iidx_v.at[j]], irows_v.at[pl.ds(j * CH, CH)], sem))
    for c in copies:
        c.wait()

    whalves = [params_v[pl.ds(0, L)], params_v[pl.ds(L, L)]]
    bias = jnp.take_along_axis(params_v[pl.ds(NF, L)],
                               jnp.zeros((L,), jnp.int32), axis=0)

    def group(g, carry):
        rows = g * L + lax.iota(jnp.int32, L)
        acc = jnp.zeros((L,), jnp.float32)
        for f in range(NF):
            fidx = jnp.full((L,), f, jnp.int32)
            pu = plsc.load_gather(urows_v, [rows, fidx])
            pi = plsc.load_gather(irows_v, [rows, fidx])
            w = jnp.take_along_axis(whalves[f // L],
                                    jnp.full((L,), f % L, jnp.int32), axis=0)
            acc = acc + pu * pi * w
        z = acc + bias
        out_v[pl.ds(g * L, L)] = 1.0 / (1.0 + jnp.exp(-z))
        return carry

    lax.fori_loop(0, bpw // L, group, 0)
    pltpu.sync_copy(out_v, out_hbm.at[pl.ds(base, bpw)])


def kernel(user_coordinates, item_coordinates, user_factors, item_factors, W1, b1):
    batch = user_coordinates.shape[0]
    assert batch % (NW * CH) == 0
    bpw = batch // NW

    params = jnp.concatenate([W1.reshape(-1), b1.reshape(-1)])
    params = jnp.pad(params, (0, NF + L - params.shape[0]))

    mesh = plsc.VectorSubcoreMesh(core_axis_name="c", subcore_axis_name="s")
    run = functools.partial(
        pl.kernel,
        mesh=mesh,
        compiler_params=pltpu.CompilerParams(needs_layout_passes=False,
                                             use_tc_tiling_on_sc=False),
        out_type=jax.ShapeDtypeStruct((batch,), jnp.float32),
        scratch_types=[
            pltpu.VMEM((bpw // CH, CH), jnp.int32),
            pltpu.VMEM((bpw // CH, CH), jnp.int32),
            pltpu.VMEM((bpw, NF), jnp.float32),
            pltpu.VMEM((bpw, NF), jnp.float32),
            pltpu.VMEM((params.shape[0],), jnp.float32),
            pltpu.VMEM((bpw,), jnp.float32),
            pltpu.SemaphoreType.DMA,
        ],
    )(functools.partial(_body, bpw=bpw))

    out = run(user_coordinates.astype(jnp.int32),
              item_coordinates.astype(jnp.int32),
              user_factors, item_factors, params)
    return out.reshape(batch, 1)
